# Initial kernel scaffold; baseline (speedup 1.0000x reference)
import jax
import jax.numpy as jnp
from jax.experimental import pallas as pl


def kernel(context):
    raise NotImplementedError("write your pallas kernel here")


# speedup vs baseline: 6.1293x; 6.1293x over previous
import jax
import jax.numpy as jnp
from jax.experimental import pallas as pl
from jax.experimental.pallas import tpu as pltpu

_L = 50
_D = 128


def _body(x_ref, o_ref):
    x = x_ref[...]
    ctx = x[:, :_L, :]
    ent = x[:, _L:, :]
    scores = jax.lax.dot_general(
        ctx, ent,
        dimension_numbers=(((2,), (2,)), ((0,), (0,))),
        preferred_element_type=jnp.float32,
    )
    o_ref[...] = jnp.sum(jnp.max(scores, axis=2), axis=1)[None, None, :]


def kernel(context):
    B, K, _, L, D = context.shape
    N = B * K
    x = context.reshape(N, 2 * L, D)
    BLK = 64
    out = pl.pallas_call(
        _body,
        grid=(N // BLK,),
        in_specs=[pl.BlockSpec((BLK, 2 * L, D), lambda i: (i, 0, 0))],
        out_specs=pl.BlockSpec((1, 1, BLK), lambda i: (i, 0, 0)),
        out_shape=jax.ShapeDtypeStruct((N // BLK, 1, BLK), jnp.float32),
    )(x)
    return out.reshape(B, K)


# speedup vs baseline: 11.7377x; 1.9150x over previous
import jax
import jax.numpy as jnp
from jax.experimental import pallas as pl
from jax.experimental.pallas import tpu as pltpu


def _body(x_ref, o_ref):
    bb, k, _, l, d = x_ref.shape
    ctx = x_ref[:, :, 0, :, :].reshape(bb * k, l, d)
    ent = x_ref[:, :, 1, :, :].reshape(bb * k, l, d)
    scores = jax.lax.dot_general(
        ctx, ent,
        dimension_numbers=(((2,), (2,)), ((0,), (0,))),
        preferred_element_type=jnp.float32,
    )
    o_ref[...] = jnp.sum(jnp.max(scores, axis=2), axis=1).reshape(bb, k)


def kernel(context):
    B, K, S, L, D = context.shape
    BLK_B = 8
    out = pl.pallas_call(
        _body,
        grid=(B // BLK_B,),
        in_specs=[pl.BlockSpec((BLK_B, K, S, L, D), lambda i: (i, 0, 0, 0, 0))],
        out_specs=pl.BlockSpec((BLK_B, K), lambda i: (i, 0)),
        out_shape=jax.ShapeDtypeStruct((B, K), jnp.float32),
    )(context)
    return out


# speedup vs baseline: 12.9102x; 1.0999x over previous
import jax
import jax.numpy as jnp
from jax.experimental import pallas as pl
from jax.experimental.pallas import tpu as pltpu


def _body(x_ref, o_ref):
    bb, k, _, l, d = x_ref.shape
    ctx = x_ref[:, :, 0, :, :].reshape(bb * k, l, d)
    ent = x_ref[:, :, 1, :, :].reshape(bb * k, l, d)
    scores = jax.lax.dot_general(
        ctx, ent,
        dimension_numbers=(((2,), (2,)), ((0,), (0,))),
        preferred_element_type=jnp.float32,
    )
    o_ref[...] = jnp.sum(jnp.max(scores, axis=2), axis=1).reshape(bb, k)


def kernel(context):
    B, K, S, L, D = context.shape
    BLK_B = 16
    out = pl.pallas_call(
        _body,
        grid=(B // BLK_B,),
        in_specs=[pl.BlockSpec((BLK_B, K, S, L, D), lambda i: (i, 0, 0, 0, 0))],
        out_specs=pl.BlockSpec((BLK_B, K), lambda i: (i, 0)),
        out_shape=jax.ShapeDtypeStruct((B, K), jnp.float32),
    )(context)
    return out


# speedup vs baseline: 13.1024x; 1.0149x over previous
import jax
import jax.numpy as jnp
from jax.experimental import pallas as pl
from jax.experimental.pallas import tpu as pltpu


def _body(x_ref, o_ref):
    bb, k, _, l, d = x_ref.shape
    ctx = x_ref[:, :, 0, :, :].reshape(bb * k, l, d)
    ent = x_ref[:, :, 1, :, :].reshape(bb * k, l, d)
    scores = jax.lax.dot_general(
        ctx, ent,
        dimension_numbers=(((2,), (2,)), ((0,), (0,))),
        preferred_element_type=jnp.float32,
    )
    o_ref[...] = jnp.sum(jnp.max(scores, axis=2), axis=1).reshape(bb, k)


def kernel(context):
    B, K, S, L, D = context.shape
    BLK_B = 32
    out = pl.pallas_call(
        _body,
        grid=(B // BLK_B,),
        in_specs=[pl.BlockSpec((BLK_B, K, S, L, D), lambda i: (i, 0, 0, 0, 0))],
        out_specs=pl.BlockSpec((BLK_B, K), lambda i: (i, 0)),
        out_shape=jax.ShapeDtypeStruct((B, K), jnp.float32),
    )(context)
    return out
